# baseline (speedup 1.0000x reference)
import functools

import jax
import jax.numpy as jnp
from jax import lax
from jax.experimental import pallas as pl
from jax.experimental.pallas import tpu as pltpu
from jax.experimental.pallas import tpu_sc as plsc


@functools.cache
def _sc_add_kernel(B, L, D):
    mesh = plsc.VectorSubcoreMesh(core_axis_name="c", subcore_axis_name="s")
    NC, NS = mesh.num_cores, mesh.num_subcores
    NW = NC * NS
    bpw = B // NW

    @functools.partial(
        pl.kernel,
        out_type=jax.ShapeDtypeStruct((B, L, D), jnp.float32),
        mesh=mesh,
        scratch_types=[
            pltpu.VMEM((L, D), jnp.float32),
            pltpu.VMEM((L, D), jnp.float32),
            pltpu.VMEM((L, D), jnp.float32),
            pltpu.VMEM((L, D), jnp.float32),
            pltpu.VMEM((L, D), jnp.float32),
            pltpu.SemaphoreType.DMA,
            pltpu.SemaphoreType.DMA,
            pltpu.SemaphoreType.DMA,
            pltpu.SemaphoreType.DMA,
            pltpu.SemaphoreType.DMA,
        ],
    )
    def k(x_hbm, pe_hbm, o_hbm, pe_v, in0, in1, out0, out1,
          sem_pe, si0, si1, so0, so1):
        wid = lax.axis_index("s") * NC + lax.axis_index("c")
        base = wid * bpw
        pltpu.async_copy(pe_hbm, pe_v, sem_pe).wait()
        pltpu.async_copy(x_hbm.at[base], in0, si0)
        pltpu.async_copy(x_hbm.at[base + 1], in1, si1)

        def add(in_v, out_v):
            @pl.loop(0, L)
            def _(r):
                for c in range(D // 16):
                    sl = pl.ds(c * 16, 16)
                    out_v[r, sl] = in_v[r, sl] + pe_v[r, sl]

        @pl.loop(0, bpw // 2)
        def _(j):
            b0 = base + 2 * j
            b1 = b0 + 1

            pltpu.make_async_copy(x_hbm.at[b0], in0, si0).wait()

            @pl.when(j > 0)
            def _():
                pltpu.make_async_copy(out0, o_hbm.at[b0 - 2], so0).wait()

            add(in0, out0)

            @pl.when(2 * j + 2 < bpw)
            def _():
                pltpu.async_copy(x_hbm.at[b0 + 2], in0, si0)

            pltpu.async_copy(out0, o_hbm.at[b0], so0)

            pltpu.make_async_copy(x_hbm.at[b1], in1, si1).wait()

            @pl.when(j > 0)
            def _():
                pltpu.make_async_copy(out1, o_hbm.at[b1 - 2], so1).wait()

            add(in1, out1)

            @pl.when(2 * j + 3 < bpw)
            def _():
                pltpu.async_copy(x_hbm.at[b1 + 2], in1, si1)

            pltpu.async_copy(out1, o_hbm.at[b1], so1)

        pltpu.make_async_copy(out0, o_hbm.at[base + bpw - 2], so0).wait()
        pltpu.make_async_copy(out1, o_hbm.at[base + bpw - 1], so1).wait()

    return k


def kernel(x, pos_embed):
    B, L, D = x.shape
    return _sc_add_kernel(B, L, D)(x, pos_embed)


# speedup vs baseline: 1.0011x; 1.0011x over previous
import functools

import jax
import jax.numpy as jnp
from jax import lax
from jax.experimental import pallas as pl
from jax.experimental.pallas import tpu as pltpu
from jax.experimental.pallas import tpu_sc as plsc

_RING = 5
_ROWS = 80


@functools.cache
def _sc_add_kernel(B, L, D):
    mesh = plsc.VectorSubcoreMesh(core_axis_name="c", subcore_axis_name="s")
    NC, NS = mesh.num_cores, mesh.num_subcores
    NW = NC * NS
    rows_total = B * L
    rpw = rows_total // NW
    N = rpw // _ROWS
    assert N % _RING == 0 and (_RING * _ROWS) % L == 0 and _ROWS % 8 == 0

    vmem = [pltpu.VMEM((_ROWS, D), jnp.float32) for _ in range(2 * _RING)]
    sems = [pltpu.SemaphoreType.DMA for _ in range(2 * _RING)]

    @functools.partial(
        pl.kernel,
        out_type=jax.ShapeDtypeStruct((rows_total, D), jnp.float32),
        mesh=mesh,
        scratch_types=[pltpu.VMEM((L, D), jnp.float32)] + vmem + sems
        + [pltpu.SemaphoreType.DMA],
    )
    def k(x_hbm, pe_hbm, o_hbm, pe_v, *rest):
        bufs_in = rest[:_RING]
        bufs_out = rest[_RING:2 * _RING]
        sin = rest[2 * _RING:3 * _RING]
        sout = rest[3 * _RING:4 * _RING]
        sem_pe = rest[4 * _RING]

        wid = lax.axis_index("s") * NC + lax.axis_index("c")
        base = wid * rpw
        pltpu.async_copy(pe_hbm, pe_v, sem_pe).wait()
        for b in range(_RING):
            pltpu.async_copy(
                x_hbm.at[pl.ds(base + b * _ROWS, _ROWS)], bufs_in[b], sin[b])

        def add(in_v, out_v, pe_off):
            w = min(_ROWS, L - pe_off)

            @pl.loop(0, w)
            def _(r):
                for c in range(D // 16):
                    sl = pl.ds(c * 16, 16)
                    out_v[r, sl] = in_v[r, sl] + pe_v[pe_off + r, sl]

            if w < _ROWS:
                @pl.loop(w, _ROWS)
                def _(r):
                    for c in range(D // 16):
                        sl = pl.ds(c * 16, 16)
                        out_v[r, sl] = in_v[r, sl] + pe_v[pe_off + r - L, sl]

        @pl.loop(0, N // _RING)
        def _(j):
            for b in range(_RING):
                c = _RING * j + b
                row0 = base + c * _ROWS
                pltpu.make_async_copy(
                    x_hbm.at[pl.ds(row0, _ROWS)], bufs_in[b], sin[b]).wait()

                @pl.when(j > 0)
                def _():
                    pltpu.make_async_copy(
                        bufs_out[b], o_hbm.at[pl.ds(row0 - _RING * _ROWS,
                                                    _ROWS)], sout[b]).wait()

                add(bufs_in[b], bufs_out[b], (b * _ROWS) % L)

                @pl.when(c + _RING < N)
                def _():
                    pltpu.async_copy(
                        x_hbm.at[pl.ds(row0 + _RING * _ROWS, _ROWS)],
                        bufs_in[b], sin[b])

                pltpu.async_copy(
                    bufs_out[b], o_hbm.at[pl.ds(row0, _ROWS)], sout[b])

        for b in range(_RING):
            row_last = base + (N - _RING + b) * _ROWS
            pltpu.make_async_copy(
                bufs_out[b], o_hbm.at[pl.ds(row_last, _ROWS)], sout[b]).wait()

    return k


def kernel(x, pos_embed):
    B, L, D = x.shape
    out = _sc_add_kernel(B, L, D)(x.reshape(B * L, D), pos_embed)
    return out.reshape(B, L, D)
